# Initial kernel scaffold; baseline (speedup 1.0000x reference)
import jax
import jax.numpy as jnp
from jax.experimental import pallas as pl


def kernel(xyz, cls_label, params):
    raise NotImplementedError("write your pallas kernel here")


# speedup vs baseline: 1.0334x; 1.0334x over previous
import functools

import jax
import jax.numpy as jnp
from jax import lax
from jax.experimental import pallas as pl


def _fps_body(xyz_ref, ion_ref, iop_ref, o_ref, *, npoint, n, nb):
    x = xyz_ref[...]
    iota_n = ion_ref[...]
    iota_p = iop_ref[...]

    def body(i, carry):
        dist, acc = carry
        fi = i.astype(jnp.float32)
        mx = jnp.max(dist, axis=1, keepdims=True)
        far = jnp.min(jnp.where(dist == mx, iota_n, jnp.float32(n)),
                      axis=1, keepdims=True)
        acc = jnp.where(iota_p == fi, jnp.broadcast_to(far, acc.shape), acc)
        onehot = (iota_n == jnp.broadcast_to(far, (nb, n))).astype(x.dtype)
        c = jnp.sum(x * onehot[:, None, :], axis=2)
        d = jnp.sum((x - c[:, :, None]) ** 2, axis=1)
        dist = jnp.minimum(dist, d)
        return dist, acc

    dist0 = jnp.full((nb, n), 1e10, dtype=jnp.float32)
    acc0 = jnp.zeros((nb, npoint), dtype=jnp.float32)
    _, acc = lax.fori_loop(0, npoint, body, (dist0, acc0))
    o_ref[...] = acc.astype(jnp.int32)


def _fps_pallas(xyz_bn3, npoint):
    b, n, _ = xyz_bn3.shape
    xt = jnp.transpose(xyz_bn3, (0, 2, 1))
    ion = jnp.broadcast_to(jnp.arange(n, dtype=jnp.float32), (b, n))
    iop = jnp.broadcast_to(jnp.arange(npoint, dtype=jnp.float32), (b, npoint))
    return pl.pallas_call(
        functools.partial(_fps_body, npoint=npoint, n=n, nb=b),
        out_shape=jax.ShapeDtypeStruct((b, npoint), jnp.int32),
    )(xt, ion, iop)


def _fps(xyz_bn3, npoint):
    x = lax.stop_gradient(xyz_bn3)
    b, n, _ = x.shape

    def step(state, _):
        distance, farthest = state
        centroid = jnp.take_along_axis(x, farthest[:, None, None], axis=1)
        dist = jnp.sum((x - centroid) ** 2, -1)
        distance = jnp.minimum(distance, dist)
        new_far = jnp.argmax(distance, -1).astype(jnp.int32)
        return (distance, new_far), farthest

    init = (jnp.full((b, n), 1e10, dtype=x.dtype),
            jnp.zeros((b,), dtype=jnp.int32))
    _, cent = jax.lax.scan(step, init, None, length=npoint)
    return jnp.transpose(cent)


def _convbn_kernel(x_ref, w_ref, b_ref, g_ref, be_ref, o_ref):
    y = jnp.dot(w_ref[...], x_ref[...], preferred_element_type=jnp.float32)
    y = y + b_ref[...]
    m = jnp.mean(y, axis=1, keepdims=True)
    yc = y - m
    v = jnp.mean(yc * yc, axis=1, keepdims=True)
    y = g_ref[...] * (yc / jnp.sqrt(v + 1e-5)) + be_ref[...]
    o_ref[...] = jnp.maximum(y, 0.0)


def _convbn(x2, layer):
    w, b, g, be = layer
    o = w.shape[0]
    m = x2.shape[1]
    return pl.pallas_call(
        _convbn_kernel,
        out_shape=jax.ShapeDtypeStruct((o, m), jnp.float32),
    )(x2, w, b[:, None], g[:, None], be[:, None])


def _bn_apply(y, m_ref, v_ref, g_ref, be_ref):
    yn = (y - m_ref[...]) / jnp.sqrt(v_ref[...] + 1e-5)
    return jnp.maximum(g_ref[...] * yn + be_ref[...], 0.0)


def _accum_stats(tid, y, s_ref, u_ref):
    @pl.when(tid == 0)
    def _():
        s_ref[...] = jnp.zeros_like(s_ref)
        u_ref[...] = jnp.zeros_like(u_ref)

    s_ref[...] += jnp.sum(y * y, axis=1, keepdims=True)
    u_ref[...] += jnp.sum(y, axis=1, keepdims=True)


def _first_kernel(x_ref, w_ref, b_ref, y_ref, s_ref, u_ref):
    y = jnp.dot(w_ref[...], x_ref[...], preferred_element_type=jnp.float32)
    y = y + b_ref[...]
    y_ref[...] = y
    _accum_stats(pl.program_id(0), y, s_ref, u_ref)


def _mid_kernel(x_ref, m_ref, v_ref, g_ref, be_ref, w_ref, b_ref,
                y_ref, s_ref, u_ref):
    x = _bn_apply(x_ref[...], m_ref, v_ref, g_ref, be_ref)
    y = jnp.dot(w_ref[...], x, preferred_element_type=jnp.float32)
    y = y + b_ref[...]
    y_ref[...] = y
    _accum_stats(pl.program_id(0), y, s_ref, u_ref)


def _finpool_kernel(x_ref, m_ref, v_ref, g_ref, be_ref, y_ref, *, k, s_cols):
    x = _bn_apply(x_ref[...], m_ref, v_ref, g_ref, be_ref)
    y_ref[...] = jnp.max(x.reshape(x.shape[0], k, s_cols), axis=1)


def _headfin_kernel(x_ref, m_ref, v_ref, g_ref, be_ref, w2_ref, b2_ref, o_ref):
    f = _bn_apply(x_ref[...], m_ref, v_ref, g_ref, be_ref)
    z = jnp.dot(w2_ref[...], f, preferred_element_type=jnp.float32)
    z = z + b2_ref[...]
    z = z - jnp.max(z, axis=0, keepdims=True)
    o_ref[...] = z - jnp.log(jnp.sum(jnp.exp(z), axis=0, keepdims=True))


def _var_kernel(x_ref, mu_ref, s_ref):
    t = pl.program_id(0)
    xc = x_ref[...] - mu_ref[...]

    @pl.when(t == 0)
    def _():
        s_ref[...] = jnp.zeros_like(s_ref)

    s_ref[...] += jnp.sum(xc * xc, axis=1, keepdims=True)


def _variance(yraw, mu, mt):
    c, m = yraw.shape
    s = pl.pallas_call(
        _var_kernel,
        grid=(m // mt,),
        in_specs=[pl.BlockSpec((c, mt), lambda t: (0, t)),
                  pl.BlockSpec((c, 1), lambda t: (0, 0))],
        out_specs=pl.BlockSpec((c, 1), lambda t: (0, 0)),
        out_shape=jax.ShapeDtypeStruct((c, 1), jnp.float32),
    )(yraw, mu)
    return s / m


def _col(a):
    return a[:, None]


def _vec_specs(shapes):
    return [pl.BlockSpec(s, lambda t: (0, 0)) for s in shapes]


def _chain_first(x2, layer, mt):
    c, m = x2.shape
    w, b = layer[0], layer[1]
    o = w.shape[0]
    y, s, u = pl.pallas_call(
        _first_kernel,
        grid=(m // mt,),
        in_specs=[pl.BlockSpec((c, mt), lambda t: (0, t))]
        + _vec_specs([(o, c), (o, 1)]),
        out_specs=[pl.BlockSpec((o, mt), lambda t: (0, t))]
        + _vec_specs([(o, 1), (o, 1)]),
        out_shape=[jax.ShapeDtypeStruct((o, m), jnp.float32),
                   jax.ShapeDtypeStruct((o, 1), jnp.float32),
                   jax.ShapeDtypeStruct((o, 1), jnp.float32)],
    )(x2, w, _col(b))
    del s
    mu = u / m
    return y, mu, _variance(y, mu, mt)


def _chain_mid(yraw, mu, v, layer_prev, layer_next, mt):
    c, m = yraw.shape
    g, be = layer_prev[2], layer_prev[3]
    w, b = layer_next[0], layer_next[1]
    o = w.shape[0]
    y, s, u = pl.pallas_call(
        _mid_kernel,
        grid=(m // mt,),
        in_specs=[pl.BlockSpec((c, mt), lambda t: (0, t))]
        + _vec_specs([(c, 1), (c, 1), (c, 1), (c, 1), (o, c), (o, 1)]),
        out_specs=[pl.BlockSpec((o, mt), lambda t: (0, t))]
        + _vec_specs([(o, 1), (o, 1)]),
        out_shape=[jax.ShapeDtypeStruct((o, m), jnp.float32),
                   jax.ShapeDtypeStruct((o, 1), jnp.float32),
                   jax.ShapeDtypeStruct((o, 1), jnp.float32)],
    )(yraw, mu, v, _col(g), _col(be), w, _col(b))
    del s
    mu2 = u / m
    return y, mu2, _variance(y, mu2, mt)


def _chain_finpool(yraw, mu, v, layer, k, s_cols):
    c, m = yraw.shape
    g, be = layer[2], layer[3]
    nb = m // (k * s_cols)
    return pl.pallas_call(
        functools.partial(_finpool_kernel, k=k, s_cols=s_cols),
        grid=(nb,),
        in_specs=[pl.BlockSpec((c, k * s_cols), lambda t: (0, t))]
        + _vec_specs([(c, 1), (c, 1), (c, 1), (c, 1)]),
        out_specs=pl.BlockSpec((c, s_cols), lambda t: (0, t)),
        out_shape=jax.ShapeDtypeStruct((c, nb * s_cols), jnp.float32),
    )(yraw, mu, v, _col(g), _col(be))


def _chain_head(yraw, mu, v, layer, w2, b2, mt):
    c, m = yraw.shape
    g, be = layer[2], layer[3]
    o2 = w2.shape[0]
    return pl.pallas_call(
        _headfin_kernel,
        grid=(m // mt,),
        in_specs=[pl.BlockSpec((c, mt), lambda t: (0, t))]
        + _vec_specs([(c, 1), (c, 1), (c, 1), (c, 1), (o2, c), (o2, 1)]),
        out_specs=pl.BlockSpec((o2, mt), lambda t: (0, t)),
        out_shape=jax.ShapeDtypeStruct((o2, m), jnp.float32),
    )(yraw, mu, v, _col(g), _col(be), w2, _col(b2))


def _sqdist(src, dst):
    d = -2.0 * jnp.matmul(src, jnp.transpose(dst, (0, 2, 1)))
    d = d + jnp.sum(src ** 2, -1)[:, :, None]
    d = d + jnp.sum(dst ** 2, -1)[:, None, :]
    return d


def _gather(points, idx):
    return jax.vmap(lambda p, i: p[i])(points, idx)


def _ball_query(radius, nsample, xyz, new_xyz):
    b, n, _ = xyz.shape
    sqr = _sqdist(new_xyz, xyz)
    idx = jnp.arange(n, dtype=jnp.int32)
    neg = jnp.where(sqr <= radius * radius, -idx, -n)
    vals, _ = lax.top_k(neg, nsample)
    gidx = -vals
    first = gidx[:, :, :1]
    return jnp.where(gidx == n, first, gidx)


def _three_nn(x1, x2):
    dists = _sqdist(x1, x2)
    negd, idx = lax.top_k(-dists, 3)
    return -negd, idx


def _set_abstraction(xyz, points, npoint, radius, nsample, layers, group_all):
    xyzt = jnp.transpose(xyz, (0, 2, 1))
    b, n, _ = xyzt.shape
    pt = jnp.transpose(points, (0, 2, 1)) if points is not None else None
    if group_all:
        new_xyz = jnp.zeros((b, 1, 3), dtype=xyzt.dtype)
        gidx = jnp.broadcast_to(jnp.arange(n, dtype=jnp.int32), (b, 1, n))
    else:
        fidx = _fps(xyzt, npoint)
        new_xyz = _gather(xyzt, fidx)
        gidx = _ball_query(radius, nsample, xyzt, new_xyz)
    grouped_xyz = _gather(xyzt, gidx) - new_xyz[:, :, None, :]
    if pt is not None:
        grouped = jnp.concatenate([grouped_xyz, _gather(pt, gidx)], axis=-1)
    else:
        grouped = grouped_xyz
    s, k, c = grouped.shape[1], grouped.shape[2], grouped.shape[3]
    x2 = jnp.transpose(grouped, (3, 0, 2, 1)).reshape(c, b * k * s)
    if group_all:
        for layer in layers:
            x2 = _convbn(x2, layer)
        o = x2.shape[0]
        pooled = jnp.max(x2.reshape(o, b, k, s), axis=2)
        pooled = jnp.transpose(pooled, (1, 0, 2))
    else:
        mt = k * s
        yraw, mu, v = _chain_first(x2, layers[0], mt)
        for lp, ln in zip(layers[:-1], layers[1:]):
            yraw, mu, v = _chain_mid(yraw, mu, v, lp, ln, mt)
        pooled = _chain_finpool(yraw, mu, v, layers[-1], k, s)
        o = pooled.shape[0]
        pooled = jnp.transpose(pooled.reshape(o, b, s), (1, 0, 2))
    return jnp.transpose(new_xyz, (0, 2, 1)), pooled


def _interpolate(xyz1, xyz2, points2):
    x1 = jnp.transpose(xyz1, (0, 2, 1))
    x2c = jnp.transpose(xyz2, (0, 2, 1))
    p2 = jnp.transpose(points2, (0, 2, 1))
    b, n, _ = x1.shape
    if x2c.shape[1] == 1:
        return jnp.tile(p2, (1, n, 1))
    d, idx = _three_nn(x1, x2c)
    recip = 1.0 / (d + 1e-8)
    norm = jnp.sum(recip, axis=2, keepdims=True)
    w = (recip / norm)[..., None]
    return jnp.sum(_gather(p2, idx) * w, axis=2)


def _feature_propagation(xyz1, xyz2, points1, points2, layers):
    interp = _interpolate(xyz1, xyz2, points2)
    if points1 is not None:
        p1 = jnp.transpose(points1, (0, 2, 1))
        new = jnp.concatenate([p1, interp], axis=-1)
    else:
        new = interp
    b, n, c = new.shape
    x2 = jnp.transpose(new, (2, 0, 1)).reshape(c, b * n)
    for layer in layers:
        x2 = _convbn(x2, layer)
    o = x2.shape[0]
    return x2.reshape(o, b, n).transpose(1, 0, 2)


def kernel(xyz, cls_label, params):
    b, _, n = xyz.shape
    l0_xyz = xyz[:, :3, :]
    l0_points = xyz[:, 3:, :]
    l1_xyz, l1_points = _set_abstraction(l0_xyz, l0_points, 512, 0.2, 32,
                                         params['sa1'], False)
    l2_xyz, l2_points = _set_abstraction(l1_xyz, l1_points, 128, 0.4, 64,
                                         params['sa2'], False)
    l3_xyz, l3_points = _set_abstraction(l2_xyz, l2_points, None, None, None,
                                         params['sa3'], True)
    l2_points = _feature_propagation(l2_xyz, l3_xyz, l2_points, l3_points,
                                     params['fp3'])
    l1_points = _feature_propagation(l1_xyz, l2_xyz, l1_points, l2_points,
                                     params['fp2'])
    interp = _interpolate(l0_xyz, l1_xyz, l1_points)
    cls_one = jnp.tile(cls_label[:, :, None], (1, 1, n))
    cat = jnp.concatenate([cls_one, l0_xyz, l0_points], axis=1)
    new = jnp.concatenate([jnp.transpose(cat, (0, 2, 1)), interp], axis=-1)
    c = new.shape[2]
    x2 = jnp.transpose(new, (2, 0, 1)).reshape(c, b * n)
    mt = x2.shape[1] // 8
    fp1 = params['fp1']
    chain = list(fp1) + [params['head1']]
    yraw, mu, v = _chain_first(x2, chain[0], mt)
    for lp, ln in zip(chain[:-1], chain[1:]):
        yraw, mu, v = _chain_mid(yraw, mu, v, lp, ln, mt)
    w2, b2 = params['conv2']
    out = _chain_head(yraw, mu, v, chain[-1], w2, b2, mt)
    return out.reshape(50, b, n).transpose(1, 2, 0)
